# Initial kernel scaffold; baseline (speedup 1.0000x reference)
import jax
import jax.numpy as jnp
from jax.experimental import pallas as pl


def kernel(x, mark, mask, mask_embedding):
    raise NotImplementedError("write your pallas kernel here")


# speedup vs baseline: 2.9092x; 2.9092x over previous
import jax
import jax.numpy as jnp
from jax.experimental import pallas as pl
from jax.experimental.pallas import tpu as pltpu

_ROWS = 2048


def _body(x_ref, m_ref, tab_ref, o_ref):
    e0 = tab_ref[0:1, :]
    e1 = tab_ref[1:2, :]
    m = m_ref[...].astype(jnp.float32)
    o_ref[...] = x_ref[...] + e0 + m * (e1 - e0)


def kernel(x, mark, mask, mask_embedding):
    del mark
    B, L, D = x.shape
    n = B * L
    xf = x.reshape(n, D)
    mf = mask.astype(jnp.int32).reshape(n, 1)
    grid = (n // _ROWS,)
    out = pl.pallas_call(
        _body,
        grid=grid,
        in_specs=[
            pl.BlockSpec((_ROWS, D), lambda i: (i, 0)),
            pl.BlockSpec((_ROWS, 1), lambda i: (i, 0)),
            pl.BlockSpec(mask_embedding.shape, lambda i: (0, 0)),
        ],
        out_specs=pl.BlockSpec((_ROWS, D), lambda i: (i, 0)),
        out_shape=jax.ShapeDtypeStruct((n, D), x.dtype),
        compiler_params=pltpu.CompilerParams(
            dimension_semantics=("arbitrary",),
        ),
    )(xf, mf, mask_embedding)
    return out.reshape(B, L, D)
